# baseline (speedup 1.0000x reference)
import functools

import jax
import jax.numpy as jnp
from jax import lax
from jax.experimental import pallas as pl
from jax.experimental.pallas import tpu as pltpu
from jax.experimental.pallas import tpu_sc as plsc

_B = 4096
_N = 200
_H = 64
_BB = 256


def _gather_pairs(weight_pairs, pair_idx):
    info = plsc.get_sparse_core_info()
    nc, ns = info.num_cores, info.num_subcores
    nw = nc * ns
    b_per_w = _B // nw
    mesh = plsc.VectorSubcoreMesh(core_axis_name="c", subcore_axis_name="s")

    @functools.partial(
        pl.kernel,
        mesh=mesh,
        out_type=jax.ShapeDtypeStruct((_B, 2 * _H), jnp.float32),
        scratch_types=[
            pltpu.VMEM((b_per_w,), jnp.int32),
            pltpu.VMEM((b_per_w, 2 * _H), jnp.float32),
            pltpu.SemaphoreType.DMA,
        ],
    )
    def gather_k(table_hbm, idx_hbm, out_hbm, idx_v, rows_v, sem):
        wid = lax.axis_index("s") * nc + lax.axis_index("c")
        base = wid * b_per_w
        pltpu.sync_copy(idx_hbm.at[pl.ds(base, b_per_w)], idx_v)
        pltpu.async_copy(table_hbm.at[idx_v], rows_v, sem).wait()
        pltpu.sync_copy(rows_v, out_hbm.at[pl.ds(base, b_per_w)])

    return gather_k(weight_pairs, pair_idx)


def _bmm(iemb, upair, parity):
    def body(x_ref, up_ref, par_ref, o_ref):
        pair = up_ref[...]
        uemb = jnp.where(par_ref[...] == 1, pair[:, _H:], pair[:, :_H])
        o_ref[...] = jnp.sum(x_ref[...] * uemb[:, None, :], axis=2)

    return pl.pallas_call(
        body,
        grid=(_B // _BB,),
        in_specs=[
            pl.BlockSpec((_BB, _N, _H), lambda i: (i, 0, 0)),
            pl.BlockSpec((_BB, 2 * _H), lambda i: (i, 0)),
            pl.BlockSpec((_BB, 1), lambda i: (i, 0)),
        ],
        out_specs=pl.BlockSpec((_BB, _N), lambda i: (i, 0)),
        out_shape=jax.ShapeDtypeStruct((_B, _N), jnp.float32),
    )(iemb, upair, parity)


def kernel(userid_input, iemb, uembedding_weight):
    idx = userid_input.reshape(-1)
    weight_pairs = uembedding_weight.reshape(-1, 2 * _H)
    upair = _gather_pairs(weight_pairs, idx // 2)
    parity = (userid_input & 1).astype(jnp.int32)
    return _bmm(iemb, upair, parity)


# speedup vs baseline: 1.3531x; 1.3531x over previous
import jax
import jax.numpy as jnp
from jax.experimental import pallas as pl
from jax.experimental.pallas import tpu as pltpu

_B = 4096
_N = 200
_H = 64
_BB = 256
_US = 4


def _body(idx_ref, iemb_ref, w_hbm, out_ref, ubuf, usem):
    i = pl.program_id(0)
    g = pl.num_programs(0)

    def ustart(step, slot):
        base = step * _BB
        for r in range(_BB):
            pltpu.make_async_copy(
                w_hbm.at[pl.ds(idx_ref[base + r], 1)],
                ubuf.at[slot, pl.ds(r, 1)],
                usem.at[slot],
            ).start()

    @pl.when(i == 0)
    def _prime():
        for k in range(_US - 1):
            ustart(k, k)

    @pl.when(i + _US - 1 < g)
    def _ahead():
        ustart(i + _US - 1, (i + _US - 1) % _US)

    pltpu.make_async_copy(
        w_hbm.at[pl.ds(0, _BB)], ubuf.at[i % _US], usem.at[i % _US]
    ).wait()

    u = ubuf[i % _US]
    out_ref[...] = jnp.sum(iemb_ref[...] * u[:, None, :], axis=2)


def kernel(userid_input, iemb, uembedding_weight):
    idx = userid_input.reshape(-1)
    grid_spec = pltpu.PrefetchScalarGridSpec(
        num_scalar_prefetch=1,
        grid=(_B // _BB,),
        in_specs=[
            pl.BlockSpec((_BB, _N, _H), lambda i, idx_ref: (i, 0, 0)),
            pl.BlockSpec(memory_space=pl.ANY),
        ],
        out_specs=pl.BlockSpec((_BB, _N), lambda i, idx_ref: (i, 0)),
        scratch_shapes=[
            pltpu.VMEM((_US, _BB, _H), jnp.float32),
            pltpu.SemaphoreType.DMA((_US,)),
        ],
    )
    return pl.pallas_call(
        _body,
        grid_spec=grid_spec,
        out_shape=jax.ShapeDtypeStruct((_B, _N), jnp.float32),
    )(idx, iemb, uembedding_weight)


# speedup vs baseline: 1.7781x; 1.3141x over previous
import jax
import jax.numpy as jnp
from jax.experimental import pallas as pl
from jax.experimental.pallas import tpu as pltpu

_B = 4096
_N = 200
_H = 64
_BB = 256
_US = 4


def _body(idx_ref, iemb_ref, oh_ref, w_hbm, out_ref, ubuf, usem):
    i = pl.program_id(0)
    g = pl.num_programs(0)

    def ustart(step, slot):
        base = step * _BB
        for r in range(_BB):
            pltpu.make_async_copy(
                w_hbm.at[pl.ds(idx_ref[base + r], 1)],
                ubuf.at[slot, pl.ds(r, 1)],
                usem.at[slot],
            ).start()

    @pl.when(i == 0)
    def _prime():
        for k in range(_US - 1):
            ustart(k, k)

    @pl.when(i + _US - 1 < g)
    def _ahead():
        ustart(i + _US - 1, (i + _US - 1) % _US)

    pltpu.make_async_copy(
        w_hbm.at[pl.ds(0, _BB)], ubuf.at[i % _US], usem.at[i % _US]
    ).wait()

    slabs = ubuf[i % _US]
    u = jnp.sum(slabs * oh_ref[...][:, :, None], axis=1)
    out_ref[...] = jnp.sum(iemb_ref[...] * u[:, None, :], axis=2)


def kernel(userid_input, iemb, uembedding_weight):
    idx = userid_input.reshape(-1)
    slab_idx = idx // 8
    onehot = (
        (idx % 8)[:, None] == jnp.arange(8, dtype=jnp.int32)[None, :]
    ).astype(jnp.float32)
    table3 = uembedding_weight.reshape(125000, 8, _H)
    grid_spec = pltpu.PrefetchScalarGridSpec(
        num_scalar_prefetch=1,
        grid=(_B // _BB,),
        in_specs=[
            pl.BlockSpec((_BB, _N, _H), lambda i, idx_ref: (i, 0, 0)),
            pl.BlockSpec((_BB, 8), lambda i, idx_ref: (i, 0)),
            pl.BlockSpec(memory_space=pl.ANY),
        ],
        out_specs=pl.BlockSpec((_BB, _N), lambda i, idx_ref: (i, 0)),
        scratch_shapes=[
            pltpu.VMEM((_US, _BB, 8, _H), jnp.float32),
            pltpu.SemaphoreType.DMA((_US,)),
        ],
    )
    return pl.pallas_call(
        _body,
        grid_spec=grid_spec,
        out_shape=jax.ShapeDtypeStruct((_B, _N), jnp.float32),
    )(slab_idx, iemb, onehot, table3)


# speedup vs baseline: 1.7891x; 1.0062x over previous
import jax
import jax.numpy as jnp
from jax.experimental import pallas as pl
from jax.experimental.pallas import tpu as pltpu

_B = 4096
_N = 200
_H = 64
_BB = 256
_US = 4
_NQ = 4


def _body(idx_ref, iemb_ref, oh_ref, w_hbm, out_ref, ubuf, usem):
    i = pl.program_id(0)
    g = pl.num_programs(0)

    def ustart(step, slot):
        base = step * _BB
        for r in range(_BB):
            pltpu.make_async_copy(
                w_hbm.at[pl.ds(idx_ref[base + r], 1)],
                ubuf.at[slot, pl.ds(r, 1)],
                usem.at[slot, r % _NQ],
            ).start()

    @pl.when(i == 0)
    def _prime():
        for k in range(_US - 1):
            ustart(k, k)

    @pl.when(i + _US - 1 < g)
    def _ahead():
        ustart(i + _US - 1, (i + _US - 1) % _US)

    for q in range(_NQ):
        pltpu.make_async_copy(
            w_hbm.at[pl.ds(0, _BB // _NQ)],
            ubuf.at[i % _US, pl.ds(0, _BB // _NQ)],
            usem.at[i % _US, q],
        ).wait()

    slabs = ubuf[i % _US]
    u = jnp.sum(slabs * oh_ref[...][:, :, None], axis=1)
    out_ref[...] = jnp.sum(iemb_ref[...] * u[:, None, :], axis=2)


def kernel(userid_input, iemb, uembedding_weight):
    idx = userid_input.reshape(-1)
    slab_idx = idx // 8
    onehot = (
        (idx % 8)[:, None] == jnp.arange(8, dtype=jnp.int32)[None, :]
    ).astype(jnp.float32)
    table3 = uembedding_weight.reshape(125000, 8, _H)
    grid_spec = pltpu.PrefetchScalarGridSpec(
        num_scalar_prefetch=1,
        grid=(_B // _BB,),
        in_specs=[
            pl.BlockSpec((_BB, _N, _H), lambda i, idx_ref: (i, 0, 0)),
            pl.BlockSpec((_BB, 8), lambda i, idx_ref: (i, 0)),
            pl.BlockSpec(memory_space=pl.ANY),
        ],
        out_specs=pl.BlockSpec((_BB, _N), lambda i, idx_ref: (i, 0)),
        scratch_shapes=[
            pltpu.VMEM((_US, _BB, 8, _H), jnp.float32),
            pltpu.SemaphoreType.DMA((_US, _NQ)),
        ],
    )
    return pl.pallas_call(
        _body,
        grid_spec=grid_spec,
        out_shape=jax.ShapeDtypeStruct((_B, _N), jnp.float32),
    )(slab_idx, iemb, onehot, table3)
